# baseline (speedup 1.0000x reference)
import jax
import jax.numpy as jnp
from jax.experimental import pallas as pl
from jax.experimental.pallas import tpu as pltpu

BATCH = 16384
MAX_LEN = 150
FLAT_ROWS = BATCH * MAX_LEN // 128
GRID = 12
BLOCK_ROWS = FLAT_ROWS // GRID


def _body(x_ref, ids_ref, mask_ref, type_ref):
    x = x_ref[...]
    ids_ref[...] = x
    mask_ref[...] = jnp.where(x == 0, 0, 1).astype(jnp.int32)
    type_ref[...] = jnp.zeros_like(x)


def kernel(inputs):
    flat = inputs.reshape(FLAT_ROWS, 128)
    spec = pl.BlockSpec((BLOCK_ROWS, 128), lambda i: (i, 0))
    out_shape = jax.ShapeDtypeStruct((FLAT_ROWS, 128), jnp.int32)
    ids, mask, type_ids = pl.pallas_call(
        _body,
        grid=(GRID,),
        in_specs=[spec],
        out_specs=[spec, spec, spec],
        out_shape=[out_shape, out_shape, out_shape],
        compiler_params=pltpu.CompilerParams(
            dimension_semantics=("arbitrary",),
        ),
    )(flat)
    shape = (BATCH, MAX_LEN)
    return (ids.reshape(shape), mask.reshape(shape), type_ids.reshape(shape))


# speedup vs baseline: 1.4655x; 1.4655x over previous
import functools

import jax
import jax.numpy as jnp
from jax import lax
from jax.experimental import pallas as pl
from jax.experimental.pallas import tpu as pltpu
from jax.experimental.pallas import tpu_sc as plsc

BATCH = 16384
MAX_LEN = 150
NW = 32
ROWS_PER_W = BATCH // NW
CHUNK_ROWS = 64
NCHUNK = ROWS_PER_W // CHUNK_ROWS
_OFFSETS = tuple(range(0, MAX_LEN - 16, 16)) + (MAX_LEN - 16,)


def _sc_body(in_hbm, ids_hbm, mask_hbm, type_hbm, ibuf, mbuf, zbuf):
    wid = lax.axis_index("s") * 2 + lax.axis_index("c")
    base = wid * ROWS_PER_W

    def zrow(r, _):
        for off in _OFFSETS:
            zbuf[r, pl.ds(off, 16)] = jnp.zeros((16,), jnp.int32)
        return 0

    lax.fori_loop(0, CHUNK_ROWS, zrow, 0, unroll=2)

    for c in range(NCHUNK):
        row0 = base + c * CHUNK_ROWS
        pltpu.sync_copy(in_hbm.at[pl.ds(row0, CHUNK_ROWS)], ibuf)
        pltpu.sync_copy(ibuf, ids_hbm.at[pl.ds(row0, CHUNK_ROWS)])
        pltpu.sync_copy(zbuf, type_hbm.at[pl.ds(row0, CHUNK_ROWS)])

        def mrow(r, _):
            for off in _OFFSETS:
                x = ibuf[r, pl.ds(off, 16)]
                mbuf[r, pl.ds(off, 16)] = jnp.where(
                    x == 0, jnp.zeros((16,), jnp.int32),
                    jnp.ones((16,), jnp.int32))
            return 0

        lax.fori_loop(0, CHUNK_ROWS, mrow, 0, unroll=2)
        pltpu.sync_copy(mbuf, mask_hbm.at[pl.ds(row0, CHUNK_ROWS)])


def kernel(inputs):
    out_t = jax.ShapeDtypeStruct((BATCH, MAX_LEN), jnp.int32)
    mesh = plsc.VectorSubcoreMesh(core_axis_name="c", subcore_axis_name="s")
    f = functools.partial(
        pl.kernel,
        mesh=mesh,
        out_type=[out_t, out_t, out_t],
        scratch_types=[
            pltpu.VMEM((CHUNK_ROWS, MAX_LEN), jnp.int32),
            pltpu.VMEM((CHUNK_ROWS, MAX_LEN), jnp.int32),
            pltpu.VMEM((CHUNK_ROWS, MAX_LEN), jnp.int32),
        ],
    )(_sc_body)
    return tuple(f(inputs))
